# Initial kernel scaffold; baseline (speedup 1.0000x reference)
import jax
import jax.numpy as jnp
from jax.experimental import pallas as pl


def kernel(x, p_pos, Wq, bq, Wk, bk, Wv, bv, Wpe1, bpe1, Wpe2, bpe2, Wg, bg, Wo, bo, scale, Wf1, bf1, Wf2, bf2, Wpool):
    raise NotImplementedError("write your pallas kernel here")


# speedup vs baseline: 12.7463x; 12.7463x over previous
import functools

import jax
import jax.numpy as jnp
from jax import lax
from jax.experimental import pallas as pl
from jax.experimental.pallas import tpu as pltpu
from jax.experimental.pallas import tpu_sc as plsc

_INV_SQRT2 = 0.7071067811865476


def _gelu(t):
    return 0.5 * t * (1.0 + lax.erf(t * _INV_SQRT2))


def _dense_body(x_ref, wqk_ref, bu_ref, wv_ref, bv_ref, wg_ref, bg_ref,
                wo_ref, bo_ref, sc_ref, wf1_ref, bf1_ref, wf2_ref, bz_ref,
                z_ref):
    xr = x_ref[...]
    u = jnp.dot(xr, wqk_ref[...], preferred_element_type=jnp.float32) + bu_ref[...]
    a = jnp.dot(u, wg_ref[...], preferred_element_type=jnp.float32) + bg_ref[...]
    a = a * (1.0 / 16.0)
    m = jnp.max(a, axis=-1, keepdims=True)
    e = jnp.exp(a - m)
    p = e / jnp.sum(e, axis=-1, keepdims=True)
    vv = jnp.dot(xr, wv_ref[...], preferred_element_type=jnp.float32) + bv_ref[...]
    r = p * vv
    x1 = jnp.dot(r, wo_ref[...], preferred_element_type=jnp.float32) + bo_ref[...] + xr
    nrm = jnp.sqrt(jnp.sum(x1 * x1, axis=-1, keepdims=True))
    y = sc_ref[...] * (x1 / (nrm * (1.0 / 16.0) + 1e-8))
    f1 = _gelu(jnp.dot(y, wf1_ref[...], preferred_element_type=jnp.float32) + bf1_ref[...])
    z_ref[...] = jnp.dot(f1, wf2_ref[...], preferred_element_type=jnp.float32) + bz_ref[...]


def _fps_body(p_ref, fx_ref, fy_ref, fz_ref):
    px = p_ref[0, 0]
    py = p_ref[0, 1]
    pz = p_ref[0, 2]
    ii = (lax.broadcasted_iota(jnp.int32, (32, 128), 0) * 128
          + lax.broadcasted_iota(jnp.int32, (32, 128), 1))
    fi = (lax.broadcasted_iota(jnp.int32, (8, 128), 0) * 128
          + lax.broadcasted_iota(jnp.int32, (8, 128), 1))

    def pick(n):
        m = ii == n
        return (jnp.sum(jnp.where(m, px, 0.0)),
                jnp.sum(jnp.where(m, py, 0.0)),
                jnp.sum(jnp.where(m, pz, 0.0)))

    def dist(sx, sy, sz):
        dx = px - sx
        dy = py - sy
        dz = pz - sz
        return dx * dx + dy * dy + dz * dz

    sx, sy, sz = pick(0)
    d = dist(sx, sy, sz)
    sel0 = fi == 0
    fx = jnp.where(sel0, sx, 0.0)
    fy = jnp.where(sel0, sy, 0.0)
    fz = jnp.where(sel0, sz, 0.0)

    def body(i, st):
        d, fx, fy, fz = st
        mx = jnp.max(d)
        nxt = jnp.min(jnp.where(d == mx, ii, jnp.int32(1 << 30)))
        sx, sy, sz = pick(nxt)
        d = jnp.minimum(d, dist(sx, sy, sz))
        sel = fi == i
        fx = jnp.where(sel, sx, fx)
        fy = jnp.where(sel, sy, fy)
        fz = jnp.where(sel, sz, fz)
        return d, fx, fy, fz

    d, fx, fy, fz = lax.fori_loop(1, 1024, body, (d, fx, fy, fz))
    fx_ref[0] = fx
    fy_ref[0] = fy
    fz_ref[0] = fz


def _knn_body(fp_ref, p_ref, wp_ref, kidx_ref, proj_ref):
    fpc = fp_ref[0]
    pm = p_ref[0]
    mat = jnp.dot(fpc, pm, preferred_element_type=jnp.float32)
    f2 = (fpc[:, 0:1] * fpc[:, 0:1] + fpc[:, 1:2] * fpc[:, 1:2]
          + fpc[:, 2:3] * fpc[:, 2:3])
    p2 = pm[0:1, :] * pm[0:1, :] + pm[1:2, :] * pm[1:2, :] + pm[2:3, :] * pm[2:3, :]
    dt = f2 + p2 - 2.0 * mat
    ji = lax.broadcasted_iota(jnp.int32, (256, 4096), 1)
    cols = []
    for _ in range(8):
        rmin = jnp.min(dt, axis=1, keepdims=True)
        am = jnp.min(jnp.where(dt == rmin, ji, jnp.int32(1 << 30)),
                     axis=1, keepdims=True)
        cols.append(am)
        dt = jnp.where(ji == am, jnp.float32(jnp.inf), dt)
    kidx_ref[0] = jnp.concatenate(cols, axis=1)
    proj_ref[0] = jnp.dot(fpc, wp_ref[...], preferred_element_type=jnp.float32)


def _pool_sc(z, idx2, fppf):
    mesh = plsc.VectorSubcoreMesh(core_axis_name="c", subcore_axis_name="s")

    @functools.partial(
        pl.kernel, mesh=mesh,
        out_type=jax.ShapeDtypeStruct((4096, 512), jnp.float32),
        scratch_types=[
            pltpu.VMEM((128, 8), jnp.int32),
            pltpu.VMEM((8, 512), jnp.float32),
            pltpu.VMEM((32, 512), jnp.float32),
            pltpu.VMEM((32, 512), jnp.float32),
            pltpu.SemaphoreType.DMA,
        ],
    )
    def run(z_hbm, idx_hbm, fpp_hbm, out_hbm, idx_v, rows_v, fpp_v, out_v, sem):
        wid = lax.axis_index("s") * 2 + lax.axis_index("c")
        base = wid * 128
        pltpu.sync_copy(idx_hbm.at[pl.ds(base, 128)], idx_v)
        for s in range(4):
            cbase = base + s * 32

            pltpu.sync_copy(fpp_hbm.at[pl.ds(cbase, 32)], fpp_v)

            def center(c, _, s=s):
                pltpu.async_copy(z_hbm.at[idx_v.at[s * 32 + c]], rows_v, sem).wait()
                for j in range(32):
                    sl = pl.ds(j * 16, 16)
                    acc = rows_v[0, sl]
                    for i in range(1, 8):
                        acc = jnp.maximum(acc, rows_v[i, sl])
                    h = acc + fpp_v[c, sl]
                    h = jnp.maximum(h, h * 0.01)
                    out_v[c, sl] = h
                return 0

            lax.fori_loop(0, 32, center, 0)
            pltpu.sync_copy(out_v, out_hbm.at[pl.ds(cbase, 32)])

    return run(z, idx2, fppf)


def kernel(x, p_pos, Wq, bq, Wk, bk, Wv, bv, Wpe1, bpe1, Wpe2, bpe2, Wg, bg,
           Wo, bo, scale, Wf1, bf1, Wf2, bf2, Wpool):
    N, B, D = x.shape
    c1 = jax.nn.gelu(bpe1, approximate=False) @ Wpe2 + bpe2
    wqk = Wq - Wk
    bu = (bq - bk + c1).reshape(1, D)
    bv2 = (bv + c1).reshape(1, D)
    wf2p = Wf2 @ Wpool[3:]
    bz = (bf2 @ Wpool[3:]).reshape(1, 2 * D)
    xr = x.reshape(N * B, D)

    z = pl.pallas_call(
        _dense_body,
        grid=(32,),
        in_specs=[
            pl.BlockSpec((512, D), lambda i: (i, 0)),
            pl.BlockSpec((D, D), lambda i: (0, 0)),
            pl.BlockSpec((1, D), lambda i: (0, 0)),
            pl.BlockSpec((D, D), lambda i: (0, 0)),
            pl.BlockSpec((1, D), lambda i: (0, 0)),
            pl.BlockSpec((D, D), lambda i: (0, 0)),
            pl.BlockSpec((1, D), lambda i: (0, 0)),
            pl.BlockSpec((D, D), lambda i: (0, 0)),
            pl.BlockSpec((1, D), lambda i: (0, 0)),
            pl.BlockSpec((1, D), lambda i: (0, 0)),
            pl.BlockSpec((D, 4 * D), lambda i: (0, 0)),
            pl.BlockSpec((1, 4 * D), lambda i: (0, 0)),
            pl.BlockSpec((4 * D, 2 * D), lambda i: (0, 0)),
            pl.BlockSpec((1, 2 * D), lambda i: (0, 0)),
        ],
        out_specs=pl.BlockSpec((512, 2 * D), lambda i: (i, 0)),
        out_shape=jax.ShapeDtypeStruct((N * B, 2 * D), jnp.float32),
    )(xr, wqk, bu, Wv, bv2, Wg, bg.reshape(1, D), Wo, bo.reshape(1, D),
      scale.reshape(1, D), Wf1, bf1.reshape(1, 4 * D), wf2p, bz)

    p3 = p_pos.transpose(0, 2, 1)
    p4 = p3.reshape(B, 3, 32, 128)
    fx, fy, fz = pl.pallas_call(
        _fps_body,
        grid=(B,),
        in_specs=[pl.BlockSpec((1, 3, 32, 128), lambda b: (b, 0, 0, 0))],
        out_specs=[pl.BlockSpec((1, 8, 128), lambda b: (b, 0, 0))] * 3,
        out_shape=[jax.ShapeDtypeStruct((B, 8, 128), jnp.float32)] * 3,
    )(p4)
    fp = jnp.stack([fx, fy, fz], axis=-1).reshape(B, 1024, 3)

    kidx, proj = pl.pallas_call(
        _knn_body,
        grid=(B, 4),
        in_specs=[
            pl.BlockSpec((1, 256, 3), lambda b, c: (b, c, 0)),
            pl.BlockSpec((1, 3, 4096), lambda b, c: (b, 0, 0)),
            pl.BlockSpec((3, 2 * D), lambda b, c: (0, 0)),
        ],
        out_specs=[
            pl.BlockSpec((1, 256, 8), lambda b, c: (b, c, 0)),
            pl.BlockSpec((1, 256, 2 * D), lambda b, c: (b, c, 0)),
        ],
        out_shape=[
            jax.ShapeDtypeStruct((B, 1024, 8), jnp.int32),
            jax.ShapeDtypeStruct((B, 1024, 2 * D), jnp.float32),
        ],
    )(fp, p3, Wpool[:3])

    idx2 = (kidx.transpose(1, 0, 2) * B
            + jnp.arange(B, dtype=jnp.int32)[None, :, None]).reshape(1024 * B, 8)
    fppf = proj.transpose(1, 0, 2).reshape(1024 * B, 2 * D)
    out = _pool_sc(z, idx2, fppf)
    return out.reshape(1024, B, 2 * D), fp


# speedup vs baseline: 34.4534x; 2.7030x over previous
import functools

import jax
import jax.numpy as jnp
from jax import lax
from jax.experimental import pallas as pl
from jax.experimental.pallas import tpu as pltpu
from jax.experimental.pallas import tpu_sc as plsc

_INV_SQRT2 = 0.7071067811865476


def _gelu(t):
    return 0.5 * t * (1.0 + lax.erf(t * _INV_SQRT2))


def _dense_body(x_ref, wqk_ref, bu_ref, wv_ref, bv_ref, wg_ref, bg_ref,
                wo_ref, bo_ref, sc_ref, wf1_ref, bf1_ref, wf2_ref, bz_ref,
                z_ref):
    xr = x_ref[...]
    u = jnp.dot(xr, wqk_ref[...], preferred_element_type=jnp.float32) + bu_ref[...]
    a = jnp.dot(u, wg_ref[...], preferred_element_type=jnp.float32) + bg_ref[...]
    a = a * (1.0 / 16.0)
    m = jnp.max(a, axis=-1, keepdims=True)
    e = jnp.exp(a - m)
    p = e / jnp.sum(e, axis=-1, keepdims=True)
    vv = jnp.dot(xr, wv_ref[...], preferred_element_type=jnp.float32) + bv_ref[...]
    r = p * vv
    x1 = jnp.dot(r, wo_ref[...], preferred_element_type=jnp.float32) + bo_ref[...] + xr
    nrm = jnp.sqrt(jnp.sum(x1 * x1, axis=-1, keepdims=True))
    y = sc_ref[...] * (x1 / (nrm * (1.0 / 16.0) + 1e-8))
    f1 = _gelu(jnp.dot(y, wf1_ref[...], preferred_element_type=jnp.float32) + bf1_ref[...])
    z_ref[...] = jnp.dot(f1, wf2_ref[...], preferred_element_type=jnp.float32) + bz_ref[...]


def _fps_body(p_ref, fx_ref, fy_ref, fz_ref):
    px = p_ref[0]
    py = p_ref[1]
    pz = p_ref[2]
    nb = px.shape[0]
    ii = (lax.broadcasted_iota(jnp.int32, (1, 32, 128), 1) * 128
          + lax.broadcasted_iota(jnp.int32, (1, 32, 128), 2))
    fi = (lax.broadcasted_iota(jnp.int32, (1, 8, 128), 1) * 128
          + lax.broadcasted_iota(jnp.int32, (1, 8, 128), 2))

    def pick(n):
        m = ii == n
        zero = jnp.zeros((), jnp.float32)
        return (jnp.sum(jnp.where(m, px, zero), axis=(1, 2), keepdims=True),
                jnp.sum(jnp.where(m, py, zero), axis=(1, 2), keepdims=True),
                jnp.sum(jnp.where(m, pz, zero), axis=(1, 2), keepdims=True))

    def dist(sx, sy, sz):
        dx = px - sx
        dy = py - sy
        dz = pz - sz
        return dx * dx + dy * dy + dz * dz

    zi = jnp.zeros((nb, 1, 1), jnp.int32)
    sx, sy, sz = pick(zi)
    d = dist(sx, sy, sz)
    sel0 = fi == 0
    fx = jnp.where(sel0, sx, jnp.zeros((nb, 8, 128), jnp.float32))
    fy = jnp.where(sel0, sy, jnp.zeros((nb, 8, 128), jnp.float32))
    fz = jnp.where(sel0, sz, jnp.zeros((nb, 8, 128), jnp.float32))

    def body(i, st):
        d, fx, fy, fz = st
        mx = jnp.max(d, axis=(1, 2), keepdims=True)
        nxt = jnp.min(jnp.where(d == mx, ii, jnp.int32(1 << 30)),
                      axis=(1, 2), keepdims=True)
        sx, sy, sz = pick(nxt)
        d = jnp.minimum(d, dist(sx, sy, sz))
        sel = fi == i
        fx = jnp.where(sel, sx, fx)
        fy = jnp.where(sel, sy, fy)
        fz = jnp.where(sel, sz, fz)
        return d, fx, fy, fz

    d, fx, fy, fz = lax.fori_loop(1, 1024, body, (d, fx, fy, fz))
    fx_ref[...] = fx
    fy_ref[...] = fy
    fz_ref[...] = fz


def _knn_body(fp_ref, p_ref, wp_ref, kidx_ref, proj_ref):
    fpc = fp_ref[0]
    pm = p_ref[0]
    mat = jnp.dot(fpc, pm, preferred_element_type=jnp.float32)
    f2 = (fpc[:, 0:1] * fpc[:, 0:1] + fpc[:, 1:2] * fpc[:, 1:2]
          + fpc[:, 2:3] * fpc[:, 2:3])
    p2 = pm[0:1, :] * pm[0:1, :] + pm[1:2, :] * pm[1:2, :] + pm[2:3, :] * pm[2:3, :]
    dt = f2 + p2 - 2.0 * mat
    ji = lax.broadcasted_iota(jnp.int32, (256, 4096), 1)
    cols = []
    for _ in range(8):
        rmin = jnp.min(dt, axis=1, keepdims=True)
        am = jnp.min(jnp.where(dt == rmin, ji, jnp.int32(1 << 30)),
                     axis=1, keepdims=True)
        cols.append(am)
        dt = jnp.where(ji == am, jnp.float32(jnp.inf), dt)
    kidx_ref[0] = jnp.concatenate(cols, axis=1)
    proj_ref[0] = jnp.dot(fpc, wp_ref[...], preferred_element_type=jnp.float32)


def _pool_sc(z, idx2, fppf):
    mesh = plsc.VectorSubcoreMesh(core_axis_name="c", subcore_axis_name="s")

    @functools.partial(
        pl.kernel, mesh=mesh,
        out_type=jax.ShapeDtypeStruct((4096, 512), jnp.float32),
        scratch_types=[
            pltpu.VMEM((128, 8), jnp.int32),
            pltpu.VMEM((8, 512), jnp.float32),
            pltpu.VMEM((32, 512), jnp.float32),
            pltpu.VMEM((32, 512), jnp.float32),
            pltpu.SemaphoreType.DMA,
        ],
    )
    def run(z_hbm, idx_hbm, fpp_hbm, out_hbm, idx_v, rows_v, fpp_v, out_v, sem):
        wid = lax.axis_index("s") * 2 + lax.axis_index("c")
        base = wid * 128
        pltpu.sync_copy(idx_hbm.at[pl.ds(base, 128)], idx_v)
        for s in range(4):
            cbase = base + s * 32

            pltpu.sync_copy(fpp_hbm.at[pl.ds(cbase, 32)], fpp_v)

            def center(c, _, s=s):
                pltpu.async_copy(z_hbm.at[idx_v.at[s * 32 + c]], rows_v, sem).wait()
                for j in range(32):
                    sl = pl.ds(j * 16, 16)
                    acc = rows_v[0, sl]
                    for i in range(1, 8):
                        acc = jnp.maximum(acc, rows_v[i, sl])
                    h = acc + fpp_v[c, sl]
                    h = jnp.maximum(h, h * 0.01)
                    out_v[c, sl] = h
                return 0

            lax.fori_loop(0, 32, center, 0)
            pltpu.sync_copy(out_v, out_hbm.at[pl.ds(cbase, 32)])

    return run(z, idx2, fppf)


def kernel(x, p_pos, Wq, bq, Wk, bk, Wv, bv, Wpe1, bpe1, Wpe2, bpe2, Wg, bg,
           Wo, bo, scale, Wf1, bf1, Wf2, bf2, Wpool):
    N, B, D = x.shape
    c1 = jax.nn.gelu(bpe1, approximate=False) @ Wpe2 + bpe2
    wqk = Wq - Wk
    bu = (bq - bk + c1).reshape(1, D)
    bv2 = (bv + c1).reshape(1, D)
    wf2p = Wf2 @ Wpool[3:]
    bz = (bf2 @ Wpool[3:]).reshape(1, 2 * D)
    xr = x.reshape(N * B, D)

    z = pl.pallas_call(
        _dense_body,
        grid=(32,),
        in_specs=[
            pl.BlockSpec((512, D), lambda i: (i, 0)),
            pl.BlockSpec((D, D), lambda i: (0, 0)),
            pl.BlockSpec((1, D), lambda i: (0, 0)),
            pl.BlockSpec((D, D), lambda i: (0, 0)),
            pl.BlockSpec((1, D), lambda i: (0, 0)),
            pl.BlockSpec((D, D), lambda i: (0, 0)),
            pl.BlockSpec((1, D), lambda i: (0, 0)),
            pl.BlockSpec((D, D), lambda i: (0, 0)),
            pl.BlockSpec((1, D), lambda i: (0, 0)),
            pl.BlockSpec((1, D), lambda i: (0, 0)),
            pl.BlockSpec((D, 4 * D), lambda i: (0, 0)),
            pl.BlockSpec((1, 4 * D), lambda i: (0, 0)),
            pl.BlockSpec((4 * D, 2 * D), lambda i: (0, 0)),
            pl.BlockSpec((1, 2 * D), lambda i: (0, 0)),
        ],
        out_specs=pl.BlockSpec((512, 2 * D), lambda i: (i, 0)),
        out_shape=jax.ShapeDtypeStruct((N * B, 2 * D), jnp.float32),
    )(xr, wqk, bu, Wv, bv2, Wg, bg.reshape(1, D), Wo, bo.reshape(1, D),
      scale.reshape(1, D), Wf1, bf1.reshape(1, 4 * D), wf2p, bz)

    p3 = p_pos.transpose(0, 2, 1)
    p4 = p_pos.transpose(2, 0, 1).reshape(3, B, 32, 128)
    fx, fy, fz = pl.pallas_call(
        _fps_body,
        out_shape=[jax.ShapeDtypeStruct((B, 8, 128), jnp.float32)] * 3,
    )(p4)
    fp = jnp.stack([fx, fy, fz], axis=-1).reshape(B, 1024, 3)

    kidx, proj = pl.pallas_call(
        _knn_body,
        grid=(B, 4),
        in_specs=[
            pl.BlockSpec((1, 256, 3), lambda b, c: (b, c, 0)),
            pl.BlockSpec((1, 3, 4096), lambda b, c: (b, 0, 0)),
            pl.BlockSpec((3, 2 * D), lambda b, c: (0, 0)),
        ],
        out_specs=[
            pl.BlockSpec((1, 256, 8), lambda b, c: (b, c, 0)),
            pl.BlockSpec((1, 256, 2 * D), lambda b, c: (b, c, 0)),
        ],
        out_shape=[
            jax.ShapeDtypeStruct((B, 1024, 8), jnp.int32),
            jax.ShapeDtypeStruct((B, 1024, 2 * D), jnp.float32),
        ],
    )(fp, p3, Wpool[:3])

    idx2 = (kidx.transpose(1, 0, 2) * B
            + jnp.arange(B, dtype=jnp.int32)[None, :, None]).reshape(1024 * B, 8)
    fppf = proj.transpose(1, 0, 2).reshape(1024 * B, 2 * D)
    out = _pool_sc(z, idx2, fppf)
    return out.reshape(1024, B, 2 * D), fp


# speedup vs baseline: 47.5506x; 1.3801x over previous
import functools

import jax
import jax.numpy as jnp
from jax import lax
from jax.experimental import pallas as pl
from jax.experimental.pallas import tpu as pltpu
from jax.experimental.pallas import tpu_sc as plsc

_INV_SQRT2 = 0.7071067811865476


def _gelu(t):
    return 0.5 * t * (1.0 + lax.erf(t * _INV_SQRT2))


def _dense_body(x_ref, wqk_ref, bu_ref, wv_ref, bv_ref, wg_ref, bg_ref,
                wo_ref, bo_ref, sc_ref, wf1_ref, bf1_ref, wf2_ref, bz_ref,
                z_ref):
    xr = x_ref[...]
    u = jnp.dot(xr, wqk_ref[...], preferred_element_type=jnp.float32) + bu_ref[...]
    a = jnp.dot(u, wg_ref[...], preferred_element_type=jnp.float32) + bg_ref[...]
    a = a * (1.0 / 16.0)
    m = jnp.max(a, axis=-1, keepdims=True)
    e = jnp.exp(a - m)
    p = e / jnp.sum(e, axis=-1, keepdims=True)
    vv = jnp.dot(xr, wv_ref[...], preferred_element_type=jnp.float32) + bv_ref[...]
    r = p * vv
    x1 = jnp.dot(r, wo_ref[...], preferred_element_type=jnp.float32) + bo_ref[...] + xr
    nrm = jnp.sqrt(jnp.sum(x1 * x1, axis=-1, keepdims=True))
    y = sc_ref[...] * (x1 / (nrm * (1.0 / 16.0) + 1e-8))
    f1 = _gelu(jnp.dot(y, wf1_ref[...], preferred_element_type=jnp.float32) + bf1_ref[...])
    z_ref[...] = jnp.dot(f1, wf2_ref[...], preferred_element_type=jnp.float32) + bz_ref[...]


def _fps_body(p_ref, fx_ref, fy_ref, fz_ref):
    px = p_ref[0]
    py = p_ref[1]
    pz = p_ref[2]
    nb = px.shape[0]
    ii = (lax.broadcasted_iota(jnp.int32, (1, 32, 128), 1) * 128
          + lax.broadcasted_iota(jnp.int32, (1, 32, 128), 2))
    fi = (lax.broadcasted_iota(jnp.int32, (1, 8, 128), 1) * 128
          + lax.broadcasted_iota(jnp.int32, (1, 8, 128), 2))
    zero = jnp.zeros((), jnp.float32)

    def sums(m):
        return (jnp.sum(jnp.where(m, px, zero), axis=(1, 2), keepdims=True),
                jnp.sum(jnp.where(m, py, zero), axis=(1, 2), keepdims=True),
                jnp.sum(jnp.where(m, pz, zero), axis=(1, 2), keepdims=True))

    def dist(sx, sy, sz):
        dx = px - sx
        dy = py - sy
        dz = pz - sz
        return dx * dx + dy * dy + dz * dz

    zi = jnp.zeros((nb, 1, 1), jnp.int32)
    sx, sy, sz = sums(ii == zi)
    d = dist(sx, sy, sz)
    sel0 = fi == 0
    fx = jnp.where(sel0, sx, jnp.zeros((nb, 8, 128), jnp.float32))
    fy = jnp.where(sel0, sy, jnp.zeros((nb, 8, 128), jnp.float32))
    fz = jnp.where(sel0, sz, jnp.zeros((nb, 8, 128), jnp.float32))

    def body(i, st):
        d, fx, fy, fz = st
        mx = jnp.max(d, axis=(1, 2), keepdims=True)
        m1 = d == mx
        sxf, syf, szf = sums(m1)
        cnt = jnp.sum(jnp.where(m1, 1.0, zero))

        def fast(_):
            return sxf, syf, szf

        def slow(_):
            nxt = jnp.min(jnp.where(m1, ii, jnp.int32(1 << 30)),
                          axis=(1, 2), keepdims=True)
            return sums(ii == nxt)

        sx, sy, sz = lax.cond(cnt == jnp.float32(nb), fast, slow, None)
        d = jnp.minimum(d, dist(sx, sy, sz))
        sel = fi == i
        fx = jnp.where(sel, sx, fx)
        fy = jnp.where(sel, sy, fy)
        fz = jnp.where(sel, sz, fz)
        return d, fx, fy, fz

    d, fx, fy, fz = lax.fori_loop(1, 1024, body, (d, fx, fy, fz))
    fx_ref[...] = fx
    fy_ref[...] = fy
    fz_ref[...] = fz


def _knn_body(fp_ref, p_ref, wp_ref, kidx_ref, proj_ref):
    fpc = fp_ref[0]
    pm = p_ref[0]
    mat = jnp.dot(fpc, pm, preferred_element_type=jnp.float32)
    f2 = (fpc[:, 0:1] * fpc[:, 0:1] + fpc[:, 1:2] * fpc[:, 1:2]
          + fpc[:, 2:3] * fpc[:, 2:3])
    p2 = pm[0:1, :] * pm[0:1, :] + pm[1:2, :] * pm[1:2, :] + pm[2:3, :] * pm[2:3, :]
    dt = f2 + p2 - 2.0 * mat
    ji = lax.broadcasted_iota(jnp.int32, (256, 4096), 1)
    cols = []
    for _ in range(8):
        rmin = jnp.min(dt, axis=1, keepdims=True)
        am = jnp.min(jnp.where(dt == rmin, ji, jnp.int32(1 << 30)),
                     axis=1, keepdims=True)
        cols.append(am)
        dt = jnp.where(ji == am, jnp.float32(jnp.inf), dt)
    kidx_ref[0] = jnp.concatenate(cols, axis=1)
    proj_ref[0] = jnp.dot(fpc, wp_ref[...], preferred_element_type=jnp.float32)


def _pool_sc(z, idx2, fppf):
    mesh = plsc.VectorSubcoreMesh(core_axis_name="c", subcore_axis_name="s")

    @functools.partial(
        pl.kernel, mesh=mesh,
        out_type=jax.ShapeDtypeStruct((4096, 512), jnp.float32),
        scratch_types=[
            pltpu.VMEM((16, 64), jnp.int32),
            pltpu.VMEM((2, 64, 512), jnp.float32),
            pltpu.VMEM((2, 8, 512), jnp.float32),
            pltpu.VMEM((2, 8, 512), jnp.float32),
            pltpu.SemaphoreType.DMA,
            pltpu.SemaphoreType.DMA,
        ],
    )
    def run(z_hbm, idx_hbm, fpp_hbm, out_hbm, idx_v, rows_v, fpp_v, out_v,
            sem_g, sem_f):
        wid = lax.axis_index("s") * 2 + lax.axis_index("c")
        base = wid * 128
        pltpu.sync_copy(idx_hbm.at[pl.ds(wid * 16, 16)], idx_v)

        def gd(g, par):
            return pltpu.make_async_copy(z_hbm.at[idx_v.at[g]],
                                         rows_v.at[par], sem_g)

        def fd(g, par):
            return pltpu.make_async_copy(fpp_hbm.at[pl.ds(base + g * 8, 8)],
                                         fpp_v.at[par], sem_f)

        def compute(g, par):
            def one(cc, _):
                for j in range(32):
                    sl = pl.ds(j * 16, 16)
                    acc = rows_v[par, cc * 8, sl]
                    for i in range(1, 8):
                        acc = jnp.maximum(acc, rows_v[par, cc * 8 + i, sl])
                    h = acc + fpp_v[par, cc, sl]
                    h = jnp.maximum(h, h * 0.01)
                    out_v[par, cc, sl] = h
                return 0

            lax.fori_loop(0, 8, one, 0)
            pltpu.sync_copy(out_v.at[par], out_hbm.at[pl.ds(base + g * 8, 8)])

        gd(0, 0).start()
        fd(0, 0).start()

        def pair(t, _):
            g0 = t * 2
            gd(g0 + 1, 1).start()
            fd(g0 + 1, 1).start()
            gd(g0, 0).wait()
            fd(g0, 0).wait()
            compute(g0, 0)

            @pl.when(t < 7)
            def _():
                gd(g0 + 2, 0).start()
                fd(g0 + 2, 0).start()

            gd(g0 + 1, 1).wait()
            fd(g0 + 1, 1).wait()
            compute(g0 + 1, 1)
            return 0

        lax.fori_loop(0, 8, pair, 0)

    return run(z, idx2.reshape(512, 64), fppf)


def kernel(x, p_pos, Wq, bq, Wk, bk, Wv, bv, Wpe1, bpe1, Wpe2, bpe2, Wg, bg,
           Wo, bo, scale, Wf1, bf1, Wf2, bf2, Wpool):
    N, B, D = x.shape
    c1 = jax.nn.gelu(bpe1, approximate=False) @ Wpe2 + bpe2
    wqk = Wq - Wk
    bu = (bq - bk + c1).reshape(1, D)
    bv2 = (bv + c1).reshape(1, D)
    wf2p = Wf2 @ Wpool[3:]
    bz = (bf2 @ Wpool[3:]).reshape(1, 2 * D)
    xr = x.reshape(N * B, D)

    z = pl.pallas_call(
        _dense_body,
        grid=(32,),
        in_specs=[
            pl.BlockSpec((512, D), lambda i: (i, 0)),
            pl.BlockSpec((D, D), lambda i: (0, 0)),
            pl.BlockSpec((1, D), lambda i: (0, 0)),
            pl.BlockSpec((D, D), lambda i: (0, 0)),
            pl.BlockSpec((1, D), lambda i: (0, 0)),
            pl.BlockSpec((D, D), lambda i: (0, 0)),
            pl.BlockSpec((1, D), lambda i: (0, 0)),
            pl.BlockSpec((D, D), lambda i: (0, 0)),
            pl.BlockSpec((1, D), lambda i: (0, 0)),
            pl.BlockSpec((1, D), lambda i: (0, 0)),
            pl.BlockSpec((D, 4 * D), lambda i: (0, 0)),
            pl.BlockSpec((1, 4 * D), lambda i: (0, 0)),
            pl.BlockSpec((4 * D, 2 * D), lambda i: (0, 0)),
            pl.BlockSpec((1, 2 * D), lambda i: (0, 0)),
        ],
        out_specs=pl.BlockSpec((512, 2 * D), lambda i: (i, 0)),
        out_shape=jax.ShapeDtypeStruct((N * B, 2 * D), jnp.float32),
    )(xr, wqk, bu, Wv, bv2, Wg, bg.reshape(1, D), Wo, bo.reshape(1, D),
      scale.reshape(1, D), Wf1, bf1.reshape(1, 4 * D), wf2p, bz)

    p3 = p_pos.transpose(0, 2, 1)
    p4 = p_pos.transpose(2, 0, 1).reshape(3, B, 32, 128)
    fx, fy, fz = pl.pallas_call(
        _fps_body,
        out_shape=[jax.ShapeDtypeStruct((B, 8, 128), jnp.float32)] * 3,
    )(p4)
    fp = jnp.stack([fx, fy, fz], axis=-1).reshape(B, 1024, 3)

    kidx, proj = pl.pallas_call(
        _knn_body,
        grid=(B, 4),
        in_specs=[
            pl.BlockSpec((1, 256, 3), lambda b, c: (b, c, 0)),
            pl.BlockSpec((1, 3, 4096), lambda b, c: (b, 0, 0)),
            pl.BlockSpec((3, 2 * D), lambda b, c: (0, 0)),
        ],
        out_specs=[
            pl.BlockSpec((1, 256, 8), lambda b, c: (b, c, 0)),
            pl.BlockSpec((1, 256, 2 * D), lambda b, c: (b, c, 0)),
        ],
        out_shape=[
            jax.ShapeDtypeStruct((B, 1024, 8), jnp.int32),
            jax.ShapeDtypeStruct((B, 1024, 2 * D), jnp.float32),
        ],
    )(fp, p3, Wpool[:3])

    idx2 = (kidx.transpose(1, 0, 2) * B
            + jnp.arange(B, dtype=jnp.int32)[None, :, None]).reshape(1024 * B, 8)
    fppf = proj.transpose(1, 0, 2).reshape(1024 * B, 2 * D)
    out = _pool_sc(z, idx2, fppf)
    return out.reshape(1024, B, 2 * D), fp


# speedup vs baseline: 47.9930x; 1.0093x over previous
import functools

import jax
import jax.numpy as jnp
from jax import lax
from jax.experimental import pallas as pl
from jax.experimental.pallas import tpu as pltpu
from jax.experimental.pallas import tpu_sc as plsc

_INV_SQRT2 = 0.7071067811865476


def _gelu(t):
    return 0.5 * t * (1.0 + lax.erf(t * _INV_SQRT2))


def _dense_body(x_ref, wqk_ref, bu_ref, wv_ref, bv_ref, wg_ref, bg_ref,
                wo_ref, bo_ref, sc_ref, wf1_ref, bf1_ref, wf2_ref, bz_ref,
                z_ref):
    xr = x_ref[...]
    u = jnp.dot(xr, wqk_ref[...], preferred_element_type=jnp.float32) + bu_ref[...]
    a = jnp.dot(u, wg_ref[...], preferred_element_type=jnp.float32) + bg_ref[...]
    a = a * (1.0 / 16.0)
    m = jnp.max(a, axis=-1, keepdims=True)
    e = jnp.exp(a - m)
    p = e / jnp.sum(e, axis=-1, keepdims=True)
    vv = jnp.dot(xr, wv_ref[...], preferred_element_type=jnp.float32) + bv_ref[...]
    r = p * vv
    x1 = jnp.dot(r, wo_ref[...], preferred_element_type=jnp.float32) + bo_ref[...] + xr
    nrm = jnp.sqrt(jnp.sum(x1 * x1, axis=-1, keepdims=True))
    y = sc_ref[...] * (x1 / (nrm * (1.0 / 16.0) + 1e-8))
    f1 = _gelu(jnp.dot(y, wf1_ref[...], preferred_element_type=jnp.float32) + bf1_ref[...])
    z_ref[...] = jnp.dot(f1, wf2_ref[...], preferred_element_type=jnp.float32) + bz_ref[...]


def _fps_body(p_ref, fx_ref, fy_ref, fz_ref):
    px = p_ref[0]
    py = p_ref[1]
    pz = p_ref[2]
    nb = px.shape[0]
    ii = (lax.broadcasted_iota(jnp.int32, (1, 32, 128), 1) * 128
          + lax.broadcasted_iota(jnp.int32, (1, 32, 128), 2))
    fi = (lax.broadcasted_iota(jnp.int32, (1, 8, 128), 1) * 128
          + lax.broadcasted_iota(jnp.int32, (1, 8, 128), 2))
    zero = jnp.zeros((), jnp.float32)

    def sums(m):
        return (jnp.sum(jnp.where(m, px, zero), axis=(1, 2), keepdims=True),
                jnp.sum(jnp.where(m, py, zero), axis=(1, 2), keepdims=True),
                jnp.sum(jnp.where(m, pz, zero), axis=(1, 2), keepdims=True))

    def dist(sx, sy, sz):
        dx = px - sx
        dy = py - sy
        dz = pz - sz
        return dx * dx + dy * dy + dz * dz

    zi = jnp.zeros((nb, 1, 1), jnp.int32)
    sx, sy, sz = sums(ii == zi)
    d = dist(sx, sy, sz)
    sel0 = fi == 0
    fx = jnp.where(sel0, sx, jnp.zeros((nb, 8, 128), jnp.float32))
    fy = jnp.where(sel0, sy, jnp.zeros((nb, 8, 128), jnp.float32))
    fz = jnp.where(sel0, sz, jnp.zeros((nb, 8, 128), jnp.float32))

    def body(i, st):
        d, fx, fy, fz = st
        mx = jnp.max(d, axis=(1, 2), keepdims=True)
        m1 = d == mx
        sxf, syf, szf = sums(m1)
        cnt = jnp.sum(jnp.where(m1, 1.0, zero))

        def fast(_):
            return sxf, syf, szf

        def slow(_):
            nxt = jnp.min(jnp.where(m1, ii, jnp.int32(1 << 30)),
                          axis=(1, 2), keepdims=True)
            return sums(ii == nxt)

        sx, sy, sz = lax.cond(cnt == jnp.float32(nb), fast, slow, None)
        d = jnp.minimum(d, dist(sx, sy, sz))
        sel = fi == i
        fx = jnp.where(sel, sx, fx)
        fy = jnp.where(sel, sy, fy)
        fz = jnp.where(sel, sz, fz)
        return d, fx, fy, fz

    def body2(t, st):
        i0 = 1 + 2 * t
        return body(i0 + 1, body(i0, st))

    st = lax.fori_loop(0, 511, body2, (d, fx, fy, fz))
    d, fx, fy, fz = body(1023, st)
    fx_ref[...] = fx
    fy_ref[...] = fy
    fz_ref[...] = fz


def _knn_body(fp_ref, p_ref, wp_ref, kidx_ref, proj_ref):
    fpc = fp_ref[0]
    pm = p_ref[0]
    mat = jnp.dot(fpc, pm, preferred_element_type=jnp.float32)
    f2 = (fpc[:, 0:1] * fpc[:, 0:1] + fpc[:, 1:2] * fpc[:, 1:2]
          + fpc[:, 2:3] * fpc[:, 2:3])
    p2 = pm[0:1, :] * pm[0:1, :] + pm[1:2, :] * pm[1:2, :] + pm[2:3, :] * pm[2:3, :]
    dt = f2 + p2 - 2.0 * mat
    ji = lax.broadcasted_iota(jnp.int32, (256, 4096), 1)
    cols = []
    for _ in range(8):
        rmin = jnp.min(dt, axis=1, keepdims=True)
        am = jnp.min(jnp.where(dt == rmin, ji, jnp.int32(1 << 30)),
                     axis=1, keepdims=True)
        cols.append(am)
        dt = jnp.where(ji == am, jnp.float32(jnp.inf), dt)
    kidx_ref[0] = jnp.concatenate(cols, axis=1)
    proj_ref[0] = jnp.dot(fpc, wp_ref[...], preferred_element_type=jnp.float32)


def _pool_sc(z, idx2, fppf):
    mesh = plsc.VectorSubcoreMesh(core_axis_name="c", subcore_axis_name="s")

    @functools.partial(
        pl.kernel, mesh=mesh,
        out_type=jax.ShapeDtypeStruct((4096, 512), jnp.float32),
        scratch_types=[
            pltpu.VMEM((16, 64), jnp.int32),
            pltpu.VMEM((2, 64, 512), jnp.float32),
            pltpu.VMEM((2, 8, 512), jnp.float32),
            pltpu.VMEM((2, 8, 512), jnp.float32),
            pltpu.SemaphoreType.DMA,
            pltpu.SemaphoreType.DMA,
        ],
    )
    def run(z_hbm, idx_hbm, fpp_hbm, out_hbm, idx_v, rows_v, fpp_v, out_v,
            sem_g, sem_f):
        wid = lax.axis_index("s") * 2 + lax.axis_index("c")
        base = wid * 128
        pltpu.sync_copy(idx_hbm.at[pl.ds(wid * 16, 16)], idx_v)

        def gd(g, par):
            return pltpu.make_async_copy(z_hbm.at[idx_v.at[g]],
                                         rows_v.at[par], sem_g)

        def fd(g, par):
            return pltpu.make_async_copy(fpp_hbm.at[pl.ds(base + g * 8, 8)],
                                         fpp_v.at[par], sem_f)

        def compute(g, par):
            def one(cc, _):
                for j in range(32):
                    sl = pl.ds(j * 16, 16)
                    acc = rows_v[par, cc * 8, sl]
                    for i in range(1, 8):
                        acc = jnp.maximum(acc, rows_v[par, cc * 8 + i, sl])
                    h = acc + fpp_v[par, cc, sl]
                    h = jnp.maximum(h, h * 0.01)
                    out_v[par, cc, sl] = h
                return 0

            lax.fori_loop(0, 8, one, 0)
            pltpu.sync_copy(out_v.at[par], out_hbm.at[pl.ds(base + g * 8, 8)])

        gd(0, 0).start()
        fd(0, 0).start()

        def pair(t, _):
            g0 = t * 2
            gd(g0 + 1, 1).start()
            fd(g0 + 1, 1).start()
            gd(g0, 0).wait()
            fd(g0, 0).wait()
            compute(g0, 0)

            @pl.when(t < 7)
            def _():
                gd(g0 + 2, 0).start()
                fd(g0 + 2, 0).start()

            gd(g0 + 1, 1).wait()
            fd(g0 + 1, 1).wait()
            compute(g0 + 1, 1)
            return 0

        lax.fori_loop(0, 8, pair, 0)

    return run(z, idx2.reshape(512, 64), fppf)


def kernel(x, p_pos, Wq, bq, Wk, bk, Wv, bv, Wpe1, bpe1, Wpe2, bpe2, Wg, bg,
           Wo, bo, scale, Wf1, bf1, Wf2, bf2, Wpool):
    N, B, D = x.shape
    c1 = jax.nn.gelu(bpe1, approximate=False) @ Wpe2 + bpe2
    wqk = Wq - Wk
    bu = (bq - bk + c1).reshape(1, D)
    bv2 = (bv + c1).reshape(1, D)
    wf2p = Wf2 @ Wpool[3:]
    bz = (bf2 @ Wpool[3:]).reshape(1, 2 * D)
    xr = x.reshape(N * B, D)

    z = pl.pallas_call(
        _dense_body,
        grid=(32,),
        in_specs=[
            pl.BlockSpec((512, D), lambda i: (i, 0)),
            pl.BlockSpec((D, D), lambda i: (0, 0)),
            pl.BlockSpec((1, D), lambda i: (0, 0)),
            pl.BlockSpec((D, D), lambda i: (0, 0)),
            pl.BlockSpec((1, D), lambda i: (0, 0)),
            pl.BlockSpec((D, D), lambda i: (0, 0)),
            pl.BlockSpec((1, D), lambda i: (0, 0)),
            pl.BlockSpec((D, D), lambda i: (0, 0)),
            pl.BlockSpec((1, D), lambda i: (0, 0)),
            pl.BlockSpec((1, D), lambda i: (0, 0)),
            pl.BlockSpec((D, 4 * D), lambda i: (0, 0)),
            pl.BlockSpec((1, 4 * D), lambda i: (0, 0)),
            pl.BlockSpec((4 * D, 2 * D), lambda i: (0, 0)),
            pl.BlockSpec((1, 2 * D), lambda i: (0, 0)),
        ],
        out_specs=pl.BlockSpec((512, 2 * D), lambda i: (i, 0)),
        out_shape=jax.ShapeDtypeStruct((N * B, 2 * D), jnp.float32),
    )(xr, wqk, bu, Wv, bv2, Wg, bg.reshape(1, D), Wo, bo.reshape(1, D),
      scale.reshape(1, D), Wf1, bf1.reshape(1, 4 * D), wf2p, bz)

    p3 = p_pos.transpose(0, 2, 1)
    p4 = p_pos.transpose(2, 0, 1).reshape(3, B, 32, 128)
    fx, fy, fz = pl.pallas_call(
        _fps_body,
        out_shape=[jax.ShapeDtypeStruct((B, 8, 128), jnp.float32)] * 3,
    )(p4)
    fp = jnp.stack([fx, fy, fz], axis=-1).reshape(B, 1024, 3)

    kidx, proj = pl.pallas_call(
        _knn_body,
        grid=(B, 4),
        in_specs=[
            pl.BlockSpec((1, 256, 3), lambda b, c: (b, c, 0)),
            pl.BlockSpec((1, 3, 4096), lambda b, c: (b, 0, 0)),
            pl.BlockSpec((3, 2 * D), lambda b, c: (0, 0)),
        ],
        out_specs=[
            pl.BlockSpec((1, 256, 8), lambda b, c: (b, c, 0)),
            pl.BlockSpec((1, 256, 2 * D), lambda b, c: (b, c, 0)),
        ],
        out_shape=[
            jax.ShapeDtypeStruct((B, 1024, 8), jnp.int32),
            jax.ShapeDtypeStruct((B, 1024, 2 * D), jnp.float32),
        ],
    )(fp, p3, Wpool[:3])

    idx2 = (kidx.transpose(1, 0, 2) * B
            + jnp.arange(B, dtype=jnp.int32)[None, :, None]).reshape(1024 * B, 8)
    fppf = proj.transpose(1, 0, 2).reshape(1024 * B, 2 * D)
    out = _pool_sc(z, idx2, fppf)
    return out.reshape(1024, B, 2 * D), fp


# speedup vs baseline: 49.4763x; 1.0309x over previous
import functools

import jax
import jax.numpy as jnp
from jax import lax
from jax.experimental import pallas as pl
from jax.experimental.pallas import tpu as pltpu
from jax.experimental.pallas import tpu_sc as plsc

_INV_SQRT2 = 0.7071067811865476


def _gelu(t):
    return 0.5 * t * (1.0 + lax.erf(t * _INV_SQRT2))


def _dense_body(x_ref, wqk_ref, bu_ref, wv_ref, bv_ref, wg_ref, bg_ref,
                wo_ref, bo_ref, sc_ref, wf1_ref, bf1_ref, wf2_ref, bz_ref,
                z_ref):
    xr = x_ref[...]
    u = jnp.dot(xr, wqk_ref[...], preferred_element_type=jnp.float32) + bu_ref[...]
    a = jnp.dot(u, wg_ref[...], preferred_element_type=jnp.float32) + bg_ref[...]
    a = a * (1.0 / 16.0)
    m = jnp.max(a, axis=-1, keepdims=True)
    e = jnp.exp(a - m)
    p = e / jnp.sum(e, axis=-1, keepdims=True)
    vv = jnp.dot(xr, wv_ref[...], preferred_element_type=jnp.float32) + bv_ref[...]
    r = p * vv
    x1 = jnp.dot(r, wo_ref[...], preferred_element_type=jnp.float32) + bo_ref[...] + xr
    nrm = jnp.sqrt(jnp.sum(x1 * x1, axis=-1, keepdims=True))
    y = sc_ref[...] * (x1 / (nrm * (1.0 / 16.0) + 1e-8))
    f1 = _gelu(jnp.dot(y, wf1_ref[...], preferred_element_type=jnp.float32) + bf1_ref[...])
    z_ref[...] = jnp.dot(f1, wf2_ref[...], preferred_element_type=jnp.float32) + bz_ref[...]


def _fps_body(p_ref, fx_ref, fy_ref, fz_ref):
    px = p_ref[0]
    py = p_ref[1]
    pz = p_ref[2]
    nb = px.shape[0]
    ii = (lax.broadcasted_iota(jnp.int32, (1, 32, 128), 1) * 128
          + lax.broadcasted_iota(jnp.int32, (1, 32, 128), 2))
    fi = (lax.broadcasted_iota(jnp.int32, (1, 8, 128), 1) * 128
          + lax.broadcasted_iota(jnp.int32, (1, 8, 128), 2))
    zero = jnp.zeros((), jnp.float32)

    def sums(m):
        return (jnp.sum(jnp.where(m, px, zero), axis=(1, 2), keepdims=True),
                jnp.sum(jnp.where(m, py, zero), axis=(1, 2), keepdims=True),
                jnp.sum(jnp.where(m, pz, zero), axis=(1, 2), keepdims=True))

    def dist(sx, sy, sz):
        dx = px - sx
        dy = py - sy
        dz = pz - sz
        return dx * dx + dy * dy + dz * dz

    zi = jnp.zeros((nb, 1, 1), jnp.int32)
    sx, sy, sz = sums(ii == zi)
    d = dist(sx, sy, sz)
    sel0 = fi == 0
    fx = jnp.where(sel0, sx, jnp.zeros((nb, 8, 128), jnp.float32))
    fy = jnp.where(sel0, sy, jnp.zeros((nb, 8, 128), jnp.float32))
    fz = jnp.where(sel0, sz, jnp.zeros((nb, 8, 128), jnp.float32))

    def body(i, st):
        d, fx, fy, fz = st
        mx = jnp.max(d, axis=(1, 2), keepdims=True)
        m1 = d == mx
        sxf, syf, szf = sums(m1)
        cnt = jnp.sum(jnp.where(m1, 1.0, zero))

        def fast(_):
            return sxf, syf, szf

        def slow(_):
            nxt = jnp.min(jnp.where(m1, ii, jnp.int32(1 << 30)),
                          axis=(1, 2), keepdims=True)
            return sums(ii == nxt)

        sx, sy, sz = lax.cond(cnt == jnp.float32(nb), fast, slow, None)
        d = jnp.minimum(d, dist(sx, sy, sz))
        sel = fi == i
        fx = jnp.where(sel, sx, fx)
        fy = jnp.where(sel, sy, fy)
        fz = jnp.where(sel, sz, fz)
        return d, fx, fy, fz

    def body2(t, st):
        i0 = 1 + 2 * t
        return body(i0 + 1, body(i0, st))

    st = lax.fori_loop(0, 511, body2, (d, fx, fy, fz))
    d, fx, fy, fz = body(1023, st)
    fx_ref[...] = fx
    fy_ref[...] = fy
    fz_ref[...] = fz


def _knn_body(fp_ref, p_ref, wp_ref, kidx_ref, proj_ref):
    fpc = fp_ref[0]
    pm = p_ref[0]
    mat = jnp.dot(fpc, pm, preferred_element_type=jnp.float32)
    f2 = (fpc[:, 0:1] * fpc[:, 0:1] + fpc[:, 1:2] * fpc[:, 1:2]
          + fpc[:, 2:3] * fpc[:, 2:3])
    p2 = pm[0:1, :] * pm[0:1, :] + pm[1:2, :] * pm[1:2, :] + pm[2:3, :] * pm[2:3, :]
    dt = f2 + p2 - 2.0 * mat
    ji = lax.broadcasted_iota(jnp.int32, (256, 4096), 1)
    cols = []
    for k in range(8):
        rmin = jnp.min(dt, axis=1, keepdims=True)
        m = dt == rmin
        am = jnp.min(jnp.where(m, ji, jnp.int32(1 << 30)),
                     axis=1, keepdims=True)
        cols.append(am)
        if k < 7:
            dt = jnp.where(m, jnp.float32(jnp.inf), dt)
    kidx_ref[0] = jnp.concatenate(cols, axis=1)
    proj_ref[0] = jnp.dot(fpc, wp_ref[...], preferred_element_type=jnp.float32)


def _pool_sc(z, idx2, fppf):
    mesh = plsc.VectorSubcoreMesh(core_axis_name="c", subcore_axis_name="s")

    @functools.partial(
        pl.kernel, mesh=mesh,
        out_type=jax.ShapeDtypeStruct((4096, 512), jnp.float32),
        scratch_types=[
            pltpu.VMEM((16, 64), jnp.int32),
            pltpu.VMEM((2, 64, 512), jnp.float32),
            pltpu.VMEM((2, 8, 512), jnp.float32),
            pltpu.VMEM((2, 8, 512), jnp.float32),
            pltpu.SemaphoreType.DMA,
            pltpu.SemaphoreType.DMA,
        ],
    )
    def run(z_hbm, idx_hbm, fpp_hbm, out_hbm, idx_v, rows_v, fpp_v, out_v,
            sem_g, sem_f):
        wid = lax.axis_index("s") * 2 + lax.axis_index("c")
        base = wid * 128
        pltpu.sync_copy(idx_hbm.at[pl.ds(wid * 16, 16)], idx_v)

        def gd(g, par):
            return pltpu.make_async_copy(z_hbm.at[idx_v.at[g]],
                                         rows_v.at[par], sem_g)

        def fd(g, par):
            return pltpu.make_async_copy(fpp_hbm.at[pl.ds(base + g * 8, 8)],
                                         fpp_v.at[par], sem_f)

        def compute(g, par):
            def one(cc, _):
                for j in range(32):
                    sl = pl.ds(j * 16, 16)
                    acc = rows_v[par, cc * 8, sl]
                    for i in range(1, 8):
                        acc = jnp.maximum(acc, rows_v[par, cc * 8 + i, sl])
                    h = acc + fpp_v[par, cc, sl]
                    h = jnp.maximum(h, h * 0.01)
                    out_v[par, cc, sl] = h
                return 0

            lax.fori_loop(0, 8, one, 0)
            pltpu.sync_copy(out_v.at[par], out_hbm.at[pl.ds(base + g * 8, 8)])

        gd(0, 0).start()
        fd(0, 0).start()

        def pair(t, _):
            g0 = t * 2
            gd(g0 + 1, 1).start()
            fd(g0 + 1, 1).start()
            gd(g0, 0).wait()
            fd(g0, 0).wait()
            compute(g0, 0)

            @pl.when(t < 7)
            def _():
                gd(g0 + 2, 0).start()
                fd(g0 + 2, 0).start()

            gd(g0 + 1, 1).wait()
            fd(g0 + 1, 1).wait()
            compute(g0 + 1, 1)
            return 0

        lax.fori_loop(0, 8, pair, 0)

    return run(z, idx2.reshape(512, 64), fppf)


def kernel(x, p_pos, Wq, bq, Wk, bk, Wv, bv, Wpe1, bpe1, Wpe2, bpe2, Wg, bg,
           Wo, bo, scale, Wf1, bf1, Wf2, bf2, Wpool):
    N, B, D = x.shape
    c1 = jax.nn.gelu(bpe1, approximate=False) @ Wpe2 + bpe2
    wqk = Wq - Wk
    bu = (bq - bk + c1).reshape(1, D)
    bv2 = (bv + c1).reshape(1, D)
    wf2p = Wf2 @ Wpool[3:]
    bz = (bf2 @ Wpool[3:]).reshape(1, 2 * D)
    xr = x.reshape(N * B, D)

    z = pl.pallas_call(
        _dense_body,
        grid=(32,),
        in_specs=[
            pl.BlockSpec((512, D), lambda i: (i, 0)),
            pl.BlockSpec((D, D), lambda i: (0, 0)),
            pl.BlockSpec((1, D), lambda i: (0, 0)),
            pl.BlockSpec((D, D), lambda i: (0, 0)),
            pl.BlockSpec((1, D), lambda i: (0, 0)),
            pl.BlockSpec((D, D), lambda i: (0, 0)),
            pl.BlockSpec((1, D), lambda i: (0, 0)),
            pl.BlockSpec((D, D), lambda i: (0, 0)),
            pl.BlockSpec((1, D), lambda i: (0, 0)),
            pl.BlockSpec((1, D), lambda i: (0, 0)),
            pl.BlockSpec((D, 4 * D), lambda i: (0, 0)),
            pl.BlockSpec((1, 4 * D), lambda i: (0, 0)),
            pl.BlockSpec((4 * D, 2 * D), lambda i: (0, 0)),
            pl.BlockSpec((1, 2 * D), lambda i: (0, 0)),
        ],
        out_specs=pl.BlockSpec((512, 2 * D), lambda i: (i, 0)),
        out_shape=jax.ShapeDtypeStruct((N * B, 2 * D), jnp.float32),
    )(xr, wqk, bu, Wv, bv2, Wg, bg.reshape(1, D), Wo, bo.reshape(1, D),
      scale.reshape(1, D), Wf1, bf1.reshape(1, 4 * D), wf2p, bz)

    p3 = p_pos.transpose(0, 2, 1)
    p4 = p_pos.transpose(2, 0, 1).reshape(3, B, 32, 128)
    fx, fy, fz = pl.pallas_call(
        _fps_body,
        out_shape=[jax.ShapeDtypeStruct((B, 8, 128), jnp.float32)] * 3,
    )(p4)
    fp = jnp.stack([fx, fy, fz], axis=-1).reshape(B, 1024, 3)

    kidx, proj = pl.pallas_call(
        _knn_body,
        grid=(B, 4),
        in_specs=[
            pl.BlockSpec((1, 256, 3), lambda b, c: (b, c, 0)),
            pl.BlockSpec((1, 3, 4096), lambda b, c: (b, 0, 0)),
            pl.BlockSpec((3, 2 * D), lambda b, c: (0, 0)),
        ],
        out_specs=[
            pl.BlockSpec((1, 256, 8), lambda b, c: (b, c, 0)),
            pl.BlockSpec((1, 256, 2 * D), lambda b, c: (b, c, 0)),
        ],
        out_shape=[
            jax.ShapeDtypeStruct((B, 1024, 8), jnp.int32),
            jax.ShapeDtypeStruct((B, 1024, 2 * D), jnp.float32),
        ],
    )(fp, p3, Wpool[:3])

    idx2 = (kidx.transpose(1, 0, 2) * B
            + jnp.arange(B, dtype=jnp.int32)[None, :, None]).reshape(1024 * B, 8)
    fppf = proj.transpose(1, 0, 2).reshape(1024 * B, 2 * D)
    out = _pool_sc(z, idx2, fppf)
    return out.reshape(1024, B, 2 * D), fp
